# baseline (speedup 1.0000x reference)
import functools

import jax
import jax.numpy as jnp
from jax import lax
from jax.experimental import pallas as pl
from jax.experimental.pallas import tpu as pltpu
from jax.experimental.pallas import tpu_sc as plsc

_L = 50
_D = 128
_SEG_PER_CHUNK = 2
_NBUF = 8
_GPC = _SEG_PER_CHUNK // 2
_GATHERS = 1
_ROWS_PER_GATHER = 100


def _sc_pool(idx2d, table, n_seg):
    info = plsc.get_sparse_core_info()
    nc, ns = info.num_cores, info.num_subcores
    nw = nc * ns
    n_chunks = idx2d.shape[0] // _GPC
    cpw = n_chunks // nw

    mesh = plsc.VectorSubcoreMesh(core_axis_name="c", subcore_axis_name="s")

    g_per_w = cpw * _GATHERS

    @functools.partial(
        pl.kernel,
        mesh=mesh,
        out_type=jax.ShapeDtypeStruct((n_seg, _D), jnp.float32),
        scratch_types=[
            pltpu.VMEM((g_per_w, _ROWS_PER_GATHER), jnp.int32),
            pltpu.VMEM((_NBUF, _GPC * _ROWS_PER_GATHER, _D), jnp.float32),
            pltpu.VMEM((_NBUF, _SEG_PER_CHUNK, _D), jnp.float32),
        ] + [pltpu.SemaphoreType.DMA] * (2 * _NBUF),
    )
    def k(table_hbm, idx_hbm, out_hbm, idx_all, rows_b, out_b, *sems):
        gsems, osems = sems[:_NBUF], sems[_NBUF:]
        wid = lax.axis_index("s") * nc + lax.axis_index("c")
        first = wid * cpw
        half = _SEG_PER_CHUNK // 2
        nb = n_seg // 2

        pltpu.sync_copy(
            idx_hbm.at[pl.ds(wid * g_per_w, g_per_w)], idx_all)

        def out_slot(chunk, k_buf):
            c0 = pltpu.make_async_copy(
                out_b.at[k_buf].at[pl.ds(0, half)],
                out_hbm.at[pl.ds(chunk * half, half)], osems[k_buf])
            c1 = pltpu.make_async_copy(
                out_b.at[k_buf].at[pl.ds(half, half)],
                out_hbm.at[pl.ds(nb + chunk * half, half)], osems[k_buf])
            return c0, c1

        def fire(c, k_buf):
            for j in range(_GPC):
                pltpu.async_copy(
                    table_hbm.at[idx_all.at[c * _GPC + j]],
                    rows_b.at[k_buf].at[pl.ds(j * _ROWS_PER_GATHER,
                                              _ROWS_PER_GATHER)],
                    gsems[k_buf])

        def process(chunk, c, k_buf):
            rows_v = rows_b.at[k_buf]
            out_v = out_b.at[k_buf]
            for j in range(_GPC):
                pltpu.make_async_copy(
                    table_hbm.at[idx_all.at[c * _GPC + j]],
                    rows_v.at[pl.ds(j * _ROWS_PER_GATHER, _ROWS_PER_GATHER)],
                    gsems[k_buf]).wait()

            @pl.when(c >= _NBUF)
            def _():
                for cp in out_slot(chunk - _NBUF, k_buf):
                    cp.wait()
            for s in range(_SEG_PER_CHUNK):
                j, h = s // 2, s % 2

                def r_body(r2, accs, j=j, h=h):
                    r = 2 * r2
                    base = j * _ROWS_PER_GATHER + h * _L
                    return tuple(
                        accs[d]
                        + (rows_v[base + r, pl.ds(d * 16, 16)]
                           + rows_v[base + r + 1, pl.ds(d * 16, 16)])
                        for d in range(8)
                    )

                accs = lax.fori_loop(
                    0, _L // 2, r_body,
                    tuple(jnp.zeros((16,), jnp.float32) for _ in range(8)),
                )
                slot = (s % 2) * half + s // 2
                for d in range(8):
                    out_v[slot, pl.ds(d * 16, 16)] = accs[d]
            pltpu.async_copy(
                out_v.at[pl.ds(0, half)],
                out_hbm.at[pl.ds(chunk * half, half)], osems[k_buf])
            pltpu.async_copy(
                out_v.at[pl.ds(half, half)],
                out_hbm.at[pl.ds(nb + chunk * half, half)], osems[k_buf])

        for c in range(_NBUF - 1):
            fire(c, c)

        def group_body(g, carry):
            base = g * _NBUF
            for kb in range(_NBUF):
                c = base + kb

                @pl.when(c + _NBUF - 1 < cpw)
                def _(c=c, kb=kb):
                    fire(c + _NBUF - 1, (kb + _NBUF - 1) % _NBUF)

                process(first + c, c, kb)
            return carry

        lax.fori_loop(0, cpw // _NBUF, group_body, 0)

        for kb in range(_NBUF):
            for cp in out_slot(first + cpw - _NBUF + kb, kb):
                cp.wait()

    return k(table, idx2d)


def _mlp_body(x0_ref, x1_ref, w1a_ref, w1b_ref, b1_ref, w2_ref, b2_ref,
              w3_ref, b3_ref, w4_ref, b4_ref, w5_ref, b5_ref, out_ref):
    x0 = x0_ref[:] * (1.0 / _L)
    x1 = x1_ref[:] * (1.0 / _L)
    h = jnp.maximum(
        jnp.dot(x0, w1a_ref[:], preferred_element_type=jnp.float32)
        + jnp.dot(x1, w1b_ref[:], preferred_element_type=jnp.float32)
        + b1_ref[:], 0.0)
    h = jnp.maximum(jnp.dot(h, w2_ref[:], preferred_element_type=jnp.float32)
                    + b2_ref[:], 0.0)
    h = jnp.maximum(jnp.dot(h, w3_ref[:], preferred_element_type=jnp.float32)
                    + b3_ref[:], 0.0)
    h = jnp.maximum(jnp.dot(h, w4_ref[:], preferred_element_type=jnp.float32)
                    + b4_ref[:], 0.0)
    o = jnp.dot(h, w5_ref[:], preferred_element_type=jnp.float32) + b5_ref[:]
    out_ref[:] = jax.nn.sigmoid(o)[:, 0]


def _mlp(sums, W1, b1, W2, b2, W3, b3, W4, b4, W5, b5):
    B = sums.shape[0] // 2
    blk = 512
    grid = (B // blk,)
    nblk = B // blk

    def full(shape):
        return pl.BlockSpec(shape, lambda i: (0, 0))

    return pl.pallas_call(
        _mlp_body,
        grid=grid,
        in_specs=[
            pl.BlockSpec((blk, _D), lambda i: (i, 0)),
            pl.BlockSpec((blk, _D), lambda i: (i + nblk, 0)),
            full((_D, 512)), full((_D, 512)), full((1, 512)),
            full(W2.shape), full((1, 256)),
            full(W3.shape), full((1, 128)),
            full(W4.shape), full((1, 64)),
            full(W5.shape), full((1, 1)),
        ],
        out_specs=pl.BlockSpec((blk,), lambda i: (i,)),
        out_shape=jax.ShapeDtypeStruct((B,), jnp.float32),
    )(sums, sums, W1[:_D], W1[_D:], b1.reshape(1, -1),
      W2, b2.reshape(1, -1), W3, b3.reshape(1, -1),
      W4, b4.reshape(1, -1), W5, b5.reshape(1, -1))


def kernel(inputs, table, W1, b1, W2, b2, W3, b3, W4, b4, W5, b5):
    B = inputs.shape[0]
    n_seg = B * 2
    idx2d = inputs.reshape(-1, _ROWS_PER_GATHER)
    sums = _sc_pool(idx2d, table, n_seg)
    return _mlp(sums, W1, b1, W2, b2, W3, b3, W4, b4, W5, b5)


# speedup vs baseline: 1.0290x; 1.0290x over previous
import functools

import jax
import jax.numpy as jnp
from jax import lax
from jax.experimental import pallas as pl
from jax.experimental.pallas import tpu as pltpu
from jax.experimental.pallas import tpu_sc as plsc

_L = 50
_D = 128
_SEG_PER_CHUNK = 4
_NBUF = 4
_GATHERS = 2
_ROWS_PER_GATHER = 100


def _sc_pool(idx2d, table, n_seg):
    info = plsc.get_sparse_core_info()
    nc, ns = info.num_cores, info.num_subcores
    nw = nc * ns
    n_chunks = idx2d.shape[0] // _GATHERS
    cpw = n_chunks // nw

    mesh = plsc.VectorSubcoreMesh(core_axis_name="c", subcore_axis_name="s")

    g_per_w = cpw * _GATHERS

    @functools.partial(
        pl.kernel,
        mesh=mesh,
        out_type=jax.ShapeDtypeStruct((n_seg, _D), jnp.float32),
        scratch_types=[
            pltpu.VMEM((g_per_w, _ROWS_PER_GATHER), jnp.int32),
            pltpu.VMEM((_NBUF, 2 * _ROWS_PER_GATHER, _D), jnp.float32),
            pltpu.VMEM((_NBUF, _SEG_PER_CHUNK, _D), jnp.float32),
        ] + [pltpu.SemaphoreType.DMA] * (2 * _NBUF),
    )
    def k(table_hbm, idx_hbm, out_hbm, idx_all, rows_b, out_b, *sems):
        gsems, osems = sems[:_NBUF], sems[_NBUF:]
        wid = lax.axis_index("s") * nc + lax.axis_index("c")
        first = wid * cpw
        half = _SEG_PER_CHUNK // 2
        nb = n_seg // 2

        pltpu.sync_copy(
            idx_hbm.at[pl.ds(wid * g_per_w, g_per_w)], idx_all)

        def out_slot(chunk, k_buf):
            c0 = pltpu.make_async_copy(
                out_b.at[k_buf].at[pl.ds(0, half)],
                out_hbm.at[pl.ds(chunk * half, half)], osems[k_buf])
            c1 = pltpu.make_async_copy(
                out_b.at[k_buf].at[pl.ds(half, half)],
                out_hbm.at[pl.ds(nb + chunk * half, half)], osems[k_buf])
            return c0, c1

        def fire(c, k_buf):
            for j in range(2):
                pltpu.async_copy(
                    table_hbm.at[idx_all.at[c * 2 + j]],
                    rows_b.at[k_buf].at[pl.ds(j * _ROWS_PER_GATHER,
                                              _ROWS_PER_GATHER)],
                    gsems[k_buf])

        def process(chunk, c, k_buf):
            rows_v = rows_b.at[k_buf]
            out_v = out_b.at[k_buf]
            for j in range(2):
                pltpu.make_async_copy(
                    table_hbm.at[idx_all.at[c * 2 + j]],
                    rows_v.at[pl.ds(j * _ROWS_PER_GATHER, _ROWS_PER_GATHER)],
                    gsems[k_buf]).wait()

            @pl.when(c >= _NBUF)
            def _():
                for cp in out_slot(chunk - _NBUF, k_buf):
                    cp.wait()
            for s in range(_SEG_PER_CHUNK):
                j, h = s // 2, s % 2

                def r_body(r2, accs, j=j, h=h):
                    r = 2 * r2
                    base = j * _ROWS_PER_GATHER + h * _L
                    return tuple(
                        accs[d]
                        + (rows_v[base + r, pl.ds(d * 16, 16)]
                           + rows_v[base + r + 1, pl.ds(d * 16, 16)])
                        for d in range(8)
                    )

                accs = lax.fori_loop(
                    0, _L // 2, r_body,
                    tuple(jnp.zeros((16,), jnp.float32) for _ in range(8)),
                )
                slot = (s % 2) * half + s // 2
                for d in range(8):
                    out_v[slot, pl.ds(d * 16, 16)] = accs[d]
            pltpu.async_copy(
                out_v.at[pl.ds(0, half)],
                out_hbm.at[pl.ds(chunk * half, half)], osems[k_buf])
            pltpu.async_copy(
                out_v.at[pl.ds(half, half)],
                out_hbm.at[pl.ds(nb + chunk * half, half)], osems[k_buf])

        for c in range(_NBUF - 1):
            fire(c, c)

        def group_body(g, carry):
            base = g * _NBUF
            for kb in range(_NBUF):
                c = base + kb

                @pl.when(c + _NBUF - 1 < cpw)
                def _(c=c, kb=kb):
                    fire(c + _NBUF - 1, (kb + _NBUF - 1) % _NBUF)

                process(first + c, c, kb)
            return carry

        lax.fori_loop(0, cpw // _NBUF, group_body, 0)

        for kb in range(_NBUF):
            for cp in out_slot(first + cpw - _NBUF + kb, kb):
                cp.wait()

    return k(table, idx2d)


def _mlp_body(x0_ref, x1_ref, w1a_ref, w1b_ref, b1_ref, w2_ref, b2_ref,
              w3_ref, b3_ref, w4_ref, b4_ref, w5_ref, b5_ref, out_ref):
    x0 = x0_ref[:] * (1.0 / _L)
    x1 = x1_ref[:] * (1.0 / _L)
    h = jnp.maximum(
        jnp.dot(x0, w1a_ref[:], preferred_element_type=jnp.float32)
        + jnp.dot(x1, w1b_ref[:], preferred_element_type=jnp.float32)
        + b1_ref[:], 0.0)
    h = jnp.maximum(jnp.dot(h, w2_ref[:], preferred_element_type=jnp.float32)
                    + b2_ref[:], 0.0)
    h = jnp.maximum(jnp.dot(h, w3_ref[:], preferred_element_type=jnp.float32)
                    + b3_ref[:], 0.0)
    h = jnp.maximum(jnp.dot(h, w4_ref[:], preferred_element_type=jnp.float32)
                    + b4_ref[:], 0.0)
    o = jnp.dot(h, w5_ref[:], preferred_element_type=jnp.float32) + b5_ref[:]
    out_ref[:] = jax.nn.sigmoid(o)[:, 0]


def _mlp(sums, W1, b1, W2, b2, W3, b3, W4, b4, W5, b5):
    B = sums.shape[0] // 2
    blk = 1024
    grid = (B // blk,)
    nblk = B // blk

    def full(shape):
        return pl.BlockSpec(shape, lambda i: (0, 0))

    return pl.pallas_call(
        _mlp_body,
        grid=grid,
        in_specs=[
            pl.BlockSpec((blk, _D), lambda i: (i, 0)),
            pl.BlockSpec((blk, _D), lambda i: (i + nblk, 0)),
            full((_D, 512)), full((_D, 512)), full((1, 512)),
            full(W2.shape), full((1, 256)),
            full(W3.shape), full((1, 128)),
            full(W4.shape), full((1, 64)),
            full(W5.shape), full((1, 1)),
        ],
        out_specs=pl.BlockSpec((blk,), lambda i: (i,)),
        out_shape=jax.ShapeDtypeStruct((B,), jnp.float32),
    )(sums, sums, W1[:_D], W1[_D:], b1.reshape(1, -1),
      W2, b2.reshape(1, -1), W3, b3.reshape(1, -1),
      W4, b4.reshape(1, -1), W5, b5.reshape(1, -1))


def kernel(inputs, table, W1, b1, W2, b2, W3, b3, W4, b4, W5, b5):
    B = inputs.shape[0]
    n_seg = B * 2
    idx2d = inputs.reshape(-1, _ROWS_PER_GATHER)
    sums = _sc_pool(idx2d, table, n_seg)
    return _mlp(sums, W1, b1, W2, b2, W3, b3, W4, b4, W5, b5)


# speedup vs baseline: 1.0362x; 1.0071x over previous
import functools

import jax
import jax.numpy as jnp
from jax import lax
from jax.experimental import pallas as pl
from jax.experimental.pallas import tpu as pltpu
from jax.experimental.pallas import tpu_sc as plsc

_L = 50
_D = 128
_SEG_PER_CHUNK = 4
_NBUF = 4
_GATHERS = 2
_ROWS_PER_GATHER = 100


def _sc_pool(idx2d, table, n_seg):
    info = plsc.get_sparse_core_info()
    nc, ns = info.num_cores, info.num_subcores
    nw = nc * ns
    n_chunks = idx2d.shape[0] // _GATHERS
    cpw = n_chunks // nw

    mesh = plsc.VectorSubcoreMesh(core_axis_name="c", subcore_axis_name="s")

    g_per_w = cpw * _GATHERS

    @functools.partial(
        pl.kernel,
        mesh=mesh,
        out_type=jax.ShapeDtypeStruct((n_seg, _D), jnp.float32),
        scratch_types=[
            pltpu.VMEM((g_per_w, _ROWS_PER_GATHER), jnp.int32),
            pltpu.VMEM((_NBUF, 2 * _ROWS_PER_GATHER, _D), jnp.float32),
            pltpu.VMEM((_NBUF, _SEG_PER_CHUNK, _D), jnp.float32),
        ] + [pltpu.SemaphoreType.DMA] * (2 * _NBUF),
    )
    def k(table_hbm, idx_hbm, out_hbm, idx_all, rows_b, out_b, *sems):
        gsems, osems = sems[:_NBUF], sems[_NBUF:]
        wid = lax.axis_index("s") * nc + lax.axis_index("c")
        first = wid * cpw
        half = _SEG_PER_CHUNK // 2
        nb = n_seg // 2

        pltpu.sync_copy(
            idx_hbm.at[pl.ds(wid * g_per_w, g_per_w)], idx_all)

        def out_slot(chunk, k_buf):
            c0 = pltpu.make_async_copy(
                out_b.at[k_buf].at[pl.ds(0, half)],
                out_hbm.at[pl.ds(chunk * half, half)], osems[k_buf])
            c1 = pltpu.make_async_copy(
                out_b.at[k_buf].at[pl.ds(half, half)],
                out_hbm.at[pl.ds(nb + chunk * half, half)], osems[k_buf])
            return c0, c1

        def fire(c, k_buf):
            for j in range(2):
                pltpu.async_copy(
                    table_hbm.at[idx_all.at[c * 2 + j]],
                    rows_b.at[k_buf].at[pl.ds(j * _ROWS_PER_GATHER,
                                              _ROWS_PER_GATHER)],
                    gsems[k_buf])

        def process(chunk, c, k_buf):
            rows_v = rows_b.at[k_buf]
            out_v = out_b.at[k_buf]
            for j in range(2):
                pltpu.make_async_copy(
                    table_hbm.at[idx_all.at[c * 2 + j]],
                    rows_v.at[pl.ds(j * _ROWS_PER_GATHER, _ROWS_PER_GATHER)],
                    gsems[k_buf]).wait()

            @pl.when(c >= _NBUF)
            def _():
                for cp in out_slot(chunk - _NBUF, k_buf):
                    cp.wait()
            for s in range(_SEG_PER_CHUNK):
                j, h = s // 2, s % 2

                def r_body(r2, accs, j=j, h=h):
                    r = 2 * r2
                    base = j * _ROWS_PER_GATHER + h * _L
                    return tuple(
                        accs[d]
                        + (rows_v[base + r, pl.ds(d * 16, 16)]
                           + rows_v[base + r + 1, pl.ds(d * 16, 16)])
                        for d in range(8)
                    )

                accs = lax.fori_loop(
                    0, _L // 2, r_body,
                    tuple(jnp.zeros((16,), jnp.float32) for _ in range(8)),
                )
                slot = (s % 2) * half + s // 2
                for d in range(8):
                    out_v[slot, pl.ds(d * 16, 16)] = accs[d]
            pltpu.async_copy(
                out_v.at[pl.ds(0, half)],
                out_hbm.at[pl.ds(chunk * half, half)], osems[k_buf])
            pltpu.async_copy(
                out_v.at[pl.ds(half, half)],
                out_hbm.at[pl.ds(nb + chunk * half, half)], osems[k_buf])

        for c in range(_NBUF - 1):
            fire(c, c)

        def group_body(g, carry):
            base = g * _NBUF
            for kb in range(_NBUF):
                c = base + kb

                @pl.when(c + _NBUF - 1 < cpw)
                def _(c=c, kb=kb):
                    fire(c + _NBUF - 1, (kb + _NBUF - 1) % _NBUF)

                process(first + c, c, kb)
            return carry

        lax.fori_loop(0, cpw // _NBUF, group_body, 0)

        for kb in range(_NBUF):
            for cp in out_slot(first + cpw - _NBUF + kb, kb):
                cp.wait()

    return k(table, idx2d)


def _mlp_body(x0_ref, x1_ref, w1a_ref, w1b_ref, b1_ref, w2_ref, b2_ref,
              w3_ref, b3_ref, w4_ref, b4_ref, w5_ref, b5_ref, out_ref):
    x0 = x0_ref[:] * (1.0 / _L)
    x1 = x1_ref[:] * (1.0 / _L)
    h = jnp.maximum(
        jnp.dot(x0, w1a_ref[:], preferred_element_type=jnp.float32)
        + jnp.dot(x1, w1b_ref[:], preferred_element_type=jnp.float32)
        + b1_ref[:], 0.0)
    h = jnp.maximum(jnp.dot(h, w2_ref[:], preferred_element_type=jnp.float32)
                    + b2_ref[:], 0.0)
    h = jnp.maximum(jnp.dot(h, w3_ref[:], preferred_element_type=jnp.float32)
                    + b3_ref[:], 0.0)
    h = jnp.maximum(jnp.dot(h, w4_ref[:], preferred_element_type=jnp.float32)
                    + b4_ref[:], 0.0)
    o = jnp.dot(h, w5_ref[:], preferred_element_type=jnp.float32) + b5_ref[:]
    out_ref[:] = jax.nn.sigmoid(o)[:, 0]


def _mlp(sums, W1, b1, W2, b2, W3, b3, W4, b4, W5, b5):
    B = sums.shape[0] // 2
    blk = 2048
    grid = (B // blk,)
    nblk = B // blk

    def full(shape):
        return pl.BlockSpec(shape, lambda i: (0, 0))

    return pl.pallas_call(
        _mlp_body,
        grid=grid,
        in_specs=[
            pl.BlockSpec((blk, _D), lambda i: (i, 0)),
            pl.BlockSpec((blk, _D), lambda i: (i + nblk, 0)),
            full((_D, 512)), full((_D, 512)), full((1, 512)),
            full(W2.shape), full((1, 256)),
            full(W3.shape), full((1, 128)),
            full(W4.shape), full((1, 64)),
            full(W5.shape), full((1, 1)),
        ],
        out_specs=pl.BlockSpec((blk,), lambda i: (i,)),
        out_shape=jax.ShapeDtypeStruct((B,), jnp.float32),
    )(sums, sums, W1[:_D], W1[_D:], b1.reshape(1, -1),
      W2, b2.reshape(1, -1), W3, b3.reshape(1, -1),
      W4, b4.reshape(1, -1), W5, b5.reshape(1, -1))


def kernel(inputs, table, W1, b1, W2, b2, W3, b3, W4, b4, W5, b5):
    B = inputs.shape[0]
    n_seg = B * 2
    idx2d = inputs.reshape(-1, _ROWS_PER_GATHER)
    sums = _sc_pool(idx2d, table, n_seg)
    return _mlp(sums, W1, b1, W2, b2, W3, b3, W4, b4, W5, b5)
